# Initial kernel scaffold; baseline (speedup 1.0000x reference)
import functools

import jax
import jax.numpy as jnp
from jax import lax
from jax.experimental import pallas as pl
from jax.experimental.pallas import tpu as pltpu
from jax.experimental.pallas import tpu_sc as plsc

N = 10000
E = 320000
D = 128
H = 128

NC = 2
NS = 16
NW = NC * NS
EPT = E // NW
CH = 80
NCH = EPT // CH
ROWS_PT = N // NS
BLK = 1000


def _mesh():
    return plsc.VectorSubcoreMesh(
        core_axis_name="c", subcore_axis_name="s", num_cores=NC, num_subcores=NS
    )


def _hist_body(src_hbm, dst_hbm, hist_hbm, srcv, dstv, hist_o, hist_i):
    cid = lax.axis_index("c")
    sid = lax.axis_index("s")
    wid = cid * NS + sid
    base = wid * EPT
    pltpu.sync_copy(src_hbm.at[pl.ds(base, EPT)], srcv)
    pltpu.sync_copy(dst_hbm.at[pl.ds(base, EPT)], dstv)

    zeros16 = jnp.zeros((16,), jnp.float32)

    def zbody(i, c):
        srcv_z = i * 16
        hist_o[pl.ds(srcv_z, 16)] = zeros16
        hist_i[pl.ds(srcv_z, 16)] = zeros16
        return c

    lax.fori_loop(0, N // 16, zbody, 0)

    ones16 = jnp.ones((16,), jnp.float32)

    def ebody(i, c):
        s = srcv[pl.ds(i * 16, 16)]
        plsc.addupdate_scatter(hist_o, [s], ones16)
        d = dstv[pl.ds(i * 16, 16)]
        plsc.addupdate_scatter(hist_i, [d], ones16)
        return c

    lax.fori_loop(0, EPT // 16, ebody, 0)

    pltpu.sync_copy(hist_o, hist_hbm.at[0, wid])
    pltpu.sync_copy(hist_i, hist_hbm.at[1, wid])


def _hist_call(src, dst):
    f = pl.kernel(
        _hist_body,
        out_type=jax.ShapeDtypeStruct((2, NW, N), jnp.float32),
        mesh=_mesh(),
        scratch_types=[
            pltpu.VMEM((EPT,), jnp.int32),
            pltpu.VMEM((EPT,), jnp.int32),
            pltpu.VMEM((N,), jnp.float32),
            pltpu.VMEM((N,), jnp.float32),
        ],
    )
    return f(src, dst)


def _scatter_body(y_hbm, srcR, dstR, z_hbm, part_hbm, srcv, dstv, rows, sem, acc):
    cid = lax.axis_index("c")
    sid = lax.axis_index("s")
    wid = cid * NS + sid
    r0 = sid * ROWS_PT
    pltpu.sync_copy(z_hbm.at[pl.ds(r0, ROWS_PT)], acc.at[pl.ds(r0, ROWS_PT)])
    pltpu.sync_copy(srcR.at[pl.ds(wid * NCH, NCH)], srcv)
    pltpu.sync_copy(dstR.at[pl.ds(wid * NCH, NCH)], dstv)
    plsc.subcore_barrier()

    def body(j, c):
        pltpu.async_copy(y_hbm.at[srcv.at[j]], rows, sem).wait()
        pltpu.sync_copy(rows, acc.at[dstv.at[j]], add=True)
        return c

    lax.fori_loop(0, NCH, body, 0)
    plsc.subcore_barrier()
    pltpu.sync_copy(acc.at[pl.ds(r0, ROWS_PT)], part_hbm.at[cid, pl.ds(r0, ROWS_PT)])


def _scatter_call(y, srcR, dstR, zrows):
    f = pl.kernel(
        _scatter_body,
        out_type=jax.ShapeDtypeStruct((NC, N, H), jnp.float32),
        mesh=_mesh(),
        scratch_types=[
            pltpu.VMEM((NCH, CH), jnp.int32),
            pltpu.VMEM((NCH, CH), jnp.int32),
            pltpu.VMEM((CH, H), jnp.float32),
            pltpu.SemaphoreType.DMA,
            pltpu.VMEM_SHARED((N, H), jnp.float32),
        ],
    )
    return f(y, srcR, dstR, zrows)


def _tc1_body(hist_ref, x_ref, w_ref, y_ref, nrm_ref):
    od = jnp.sum(hist_ref[0], axis=-1, keepdims=True)
    idg = jnp.sum(hist_ref[1], axis=-1, keepdims=True)
    onrm = lax.rsqrt(jnp.maximum(od, 1.0))
    inrm = lax.rsqrt(jnp.maximum(idg, 1.0))
    y = jnp.dot(x_ref[...], w_ref[...], preferred_element_type=jnp.float32)
    y_ref[...] = onrm * y
    nrm_ref[...] = jnp.concatenate([onrm, inrm], axis=1)


def _tc1_call(hist_t, x, w1):
    return pl.pallas_call(
        _tc1_body,
        grid=(N // BLK,),
        in_specs=[
            pl.BlockSpec((2, BLK, NW), lambda i: (0, i, 0)),
            pl.BlockSpec((BLK, D), lambda i: (i, 0)),
            pl.BlockSpec((D, H), lambda i: (0, 0)),
        ],
        out_specs=[
            pl.BlockSpec((BLK, H), lambda i: (i, 0)),
            pl.BlockSpec((BLK, 2), lambda i: (i, 0)),
        ],
        out_shape=[
            jax.ShapeDtypeStruct((N, H), jnp.float32),
            jax.ShapeDtypeStruct((N, 2), jnp.float32),
        ],
    )(hist_t, x, w1)


def _tc2_body(part_ref, nrm_ref, b_ref, w_ref, y_ref):
    agg = part_ref[0] + part_ref[1]
    inrm = nrm_ref[:, 1:2]
    onrm = nrm_ref[:, 0:1]
    h = jnp.maximum(agg * inrm + b_ref[...], 0.0)
    y_ref[...] = onrm * jnp.dot(h, w_ref[...], preferred_element_type=jnp.float32)


def _tc2_call(part, nrm, b1, w2):
    return pl.pallas_call(
        _tc2_body,
        grid=(N // BLK,),
        in_specs=[
            pl.BlockSpec((NC, BLK, H), lambda i: (0, i, 0)),
            pl.BlockSpec((BLK, 2), lambda i: (i, 0)),
            pl.BlockSpec((1, H), lambda i: (0, 0)),
            pl.BlockSpec((H, H), lambda i: (0, 0)),
        ],
        out_specs=pl.BlockSpec((BLK, H), lambda i: (i, 0)),
        out_shape=jax.ShapeDtypeStruct((N, H), jnp.float32),
    )(part, nrm, b1, w2)


def _tc3_body(part_ref, nrm_ref, b_ref, o_ref):
    agg = part_ref[0] + part_ref[1]
    inrm = nrm_ref[:, 1:2]
    o_ref[...] = jnp.maximum(agg * inrm + b_ref[...], 0.0)


def _tc3_call(part, nrm, b2):
    return pl.pallas_call(
        _tc3_body,
        grid=(N // BLK,),
        in_specs=[
            pl.BlockSpec((NC, BLK, H), lambda i: (0, i, 0)),
            pl.BlockSpec((BLK, 2), lambda i: (i, 0)),
            pl.BlockSpec((1, H), lambda i: (0, 0)),
        ],
        out_specs=pl.BlockSpec((BLK, H), lambda i: (i, 0)),
        out_shape=jax.ShapeDtypeStruct((N, H), jnp.float32),
    )(part, nrm, b2)


def kernel(features, edge_index, W1, b1, W2, b2):
    src = edge_index[0]
    dst = edge_index[1]
    srcR = src.reshape(E // CH, CH)
    dstR = dst.reshape(E // CH, CH)
    zrows = jnp.zeros((N, H), jnp.float32)

    hist = _hist_call(src, dst)
    hist_t = jnp.transpose(hist, (0, 2, 1))
    y1, nrm = _tc1_call(hist_t, features, W1)
    part1 = _scatter_call(y1, srcR, dstR, zrows)
    y2 = _tc2_call(part1, nrm, b1.reshape(1, H), W2)
    part2 = _scatter_call(y2, srcR, dstR, zrows)
    out = _tc3_call(part2, nrm, b2.reshape(1, H))
    return out


# speedup vs baseline: 7.9619x; 7.9619x over previous
import functools

import jax
import jax.numpy as jnp
from jax import lax
from jax.experimental import pallas as pl
from jax.experimental.pallas import tpu as pltpu
from jax.experimental.pallas import tpu_sc as plsc

N = 10000
E = 320000
D = 128
H = 128

NC = 2
NS = 16
NW = NC * NS
EPT = E // NW
CH = 80
NCH = EPT // CH
ROWS_PT = N // NS
BLK = 1000


def _mesh():
    return plsc.VectorSubcoreMesh(
        core_axis_name="c", subcore_axis_name="s", num_cores=NC, num_subcores=NS
    )


def _hist_body(src_hbm, dst_hbm, hist_hbm, srcv, dstv, hist_o, hist_i):
    cid = lax.axis_index("c")
    sid = lax.axis_index("s")
    wid = cid * NS + sid
    base = wid * EPT
    pltpu.sync_copy(src_hbm.at[pl.ds(base, EPT)], srcv)
    pltpu.sync_copy(dst_hbm.at[pl.ds(base, EPT)], dstv)

    zeros16 = jnp.zeros((16,), jnp.float32)

    def zbody(i, c):
        srcv_z = i * 16
        hist_o[pl.ds(srcv_z, 16)] = zeros16
        hist_i[pl.ds(srcv_z, 16)] = zeros16
        return c

    lax.fori_loop(0, N // 16, zbody, 0)

    ones16 = jnp.ones((16,), jnp.float32)

    def ebody(i, c):
        s = srcv[pl.ds(i * 16, 16)]
        plsc.addupdate_scatter(hist_o, [s], ones16)
        d = dstv[pl.ds(i * 16, 16)]
        plsc.addupdate_scatter(hist_i, [d], ones16)
        return c

    lax.fori_loop(0, EPT // 16, ebody, 0)

    pltpu.sync_copy(hist_o, hist_hbm.at[0, wid])
    pltpu.sync_copy(hist_i, hist_hbm.at[1, wid])


def _hist_call(src, dst):
    f = pl.kernel(
        _hist_body,
        out_type=jax.ShapeDtypeStruct((2, NW, N), jnp.float32),
        mesh=_mesh(),
        scratch_types=[
            pltpu.VMEM((EPT,), jnp.int32),
            pltpu.VMEM((EPT,), jnp.int32),
            pltpu.VMEM((N,), jnp.float32),
            pltpu.VMEM((N,), jnp.float32),
        ],
        compiler_params=pltpu.CompilerParams(needs_layout_passes=False),
    )
    return f(src, dst)


def _scatter_body(y_hbm, srcR, dstR, z_hbm, part_hbm, srcv, dstv, rows, sem, acc):
    cid = lax.axis_index("c")
    sid = lax.axis_index("s")
    wid = cid * NS + sid
    r0 = sid * ROWS_PT
    pltpu.sync_copy(z_hbm.at[pl.ds(r0, ROWS_PT)], acc.at[pl.ds(r0, ROWS_PT)])
    pltpu.sync_copy(srcR.at[pl.ds(wid * NCH, NCH)], srcv)
    pltpu.sync_copy(dstR.at[pl.ds(wid * NCH, NCH)], dstv)
    plsc.subcore_barrier()

    def body(j, c):
        pltpu.async_copy(y_hbm.at[srcv.at[j]], rows, sem).wait()
        pltpu.sync_copy(rows, acc.at[dstv.at[j]], add=True)
        return c

    lax.fori_loop(0, NCH, body, 0)
    plsc.subcore_barrier()
    pltpu.sync_copy(acc.at[pl.ds(r0, ROWS_PT)], part_hbm.at[cid, pl.ds(r0, ROWS_PT)])


def _scatter_call(y, srcR, dstR, zrows):
    f = pl.kernel(
        _scatter_body,
        out_type=jax.ShapeDtypeStruct((NC, N, H), jnp.float32),
        mesh=_mesh(),
        scratch_types=[
            pltpu.VMEM((NCH, CH), jnp.int32),
            pltpu.VMEM((NCH, CH), jnp.int32),
            pltpu.VMEM((CH, H), jnp.float32),
            pltpu.SemaphoreType.DMA,
            pltpu.VMEM_SHARED((N, H), jnp.float32),
        ],
        compiler_params=pltpu.CompilerParams(
            needs_layout_passes=False, use_tc_tiling_on_sc=False
        ),
    )
    return f(y, srcR, dstR, zrows)


def _tc1_body(hist_ref, x_ref, w_ref, y_ref, nrm_ref):
    od = jnp.sum(hist_ref[0], axis=-1, keepdims=True)
    idg = jnp.sum(hist_ref[1], axis=-1, keepdims=True)
    onrm = lax.rsqrt(jnp.maximum(od, 1.0))
    inrm = lax.rsqrt(jnp.maximum(idg, 1.0))
    y = jnp.dot(x_ref[...], w_ref[...], preferred_element_type=jnp.float32)
    y_ref[...] = onrm * y
    nrm_ref[...] = jnp.concatenate([onrm, inrm], axis=1)


def _tc1_call(hist_t, x, w1):
    return pl.pallas_call(
        _tc1_body,
        grid=(N // BLK,),
        in_specs=[
            pl.BlockSpec((2, BLK, NW), lambda i: (0, i, 0)),
            pl.BlockSpec((BLK, D), lambda i: (i, 0)),
            pl.BlockSpec((D, H), lambda i: (0, 0)),
        ],
        out_specs=[
            pl.BlockSpec((BLK, H), lambda i: (i, 0)),
            pl.BlockSpec((BLK, 2), lambda i: (i, 0)),
        ],
        out_shape=[
            jax.ShapeDtypeStruct((N, H), jnp.float32),
            jax.ShapeDtypeStruct((N, 2), jnp.float32),
        ],
    )(hist_t, x, w1)


def _tc2_body(part_ref, nrm_ref, b_ref, w_ref, y_ref):
    agg = part_ref[0] + part_ref[1]
    inrm = nrm_ref[:, 1:2]
    onrm = nrm_ref[:, 0:1]
    h = jnp.maximum(agg * inrm + b_ref[...], 0.0)
    y_ref[...] = onrm * jnp.dot(h, w_ref[...], preferred_element_type=jnp.float32)


def _tc2_call(part, nrm, b1, w2):
    return pl.pallas_call(
        _tc2_body,
        grid=(N // BLK,),
        in_specs=[
            pl.BlockSpec((NC, BLK, H), lambda i: (0, i, 0)),
            pl.BlockSpec((BLK, 2), lambda i: (i, 0)),
            pl.BlockSpec((1, H), lambda i: (0, 0)),
            pl.BlockSpec((H, H), lambda i: (0, 0)),
        ],
        out_specs=pl.BlockSpec((BLK, H), lambda i: (i, 0)),
        out_shape=jax.ShapeDtypeStruct((N, H), jnp.float32),
    )(part, nrm, b1, w2)


def _tc3_body(part_ref, nrm_ref, b_ref, o_ref):
    agg = part_ref[0] + part_ref[1]
    inrm = nrm_ref[:, 1:2]
    o_ref[...] = jnp.maximum(agg * inrm + b_ref[...], 0.0)


def _tc3_call(part, nrm, b2):
    return pl.pallas_call(
        _tc3_body,
        grid=(N // BLK,),
        in_specs=[
            pl.BlockSpec((NC, BLK, H), lambda i: (0, i, 0)),
            pl.BlockSpec((BLK, 2), lambda i: (i, 0)),
            pl.BlockSpec((1, H), lambda i: (0, 0)),
        ],
        out_specs=pl.BlockSpec((BLK, H), lambda i: (i, 0)),
        out_shape=jax.ShapeDtypeStruct((N, H), jnp.float32),
    )(part, nrm, b2)


def kernel(features, edge_index, W1, b1, W2, b2):
    src = edge_index[0]
    dst = edge_index[1]
    srcR = src.reshape(E // CH, CH)
    dstR = dst.reshape(E // CH, CH)
    zrows = jnp.zeros((N, H), jnp.float32)

    hist = _hist_call(src, dst)
    hist_t = jnp.transpose(hist, (0, 2, 1))
    y1, nrm = _tc1_call(hist_t, features, W1)
    part1 = _scatter_call(y1, srcR, dstR, zrows)
    y2 = _tc2_call(part1, nrm, b1.reshape(1, H), W2)
    part2 = _scatter_call(y2, srcR, dstR, zrows)
    out = _tc3_call(part2, nrm, b2.reshape(1, H))
    return out


# speedup vs baseline: 14.4543x; 1.8154x over previous
import functools

import jax
import jax.numpy as jnp
from jax import lax
from jax.experimental import pallas as pl
from jax.experimental.pallas import tpu as pltpu
from jax.experimental.pallas import tpu_sc as plsc

N = 10000
E = 320000
D = 128
H = 128

NC = 2
NS = 16
NW = NC * NS
EPT = E // NW
CH = 40
NCH = EPT // CH
ROWS_PT = N // NS
BLK = 1000


def _mesh():
    return plsc.VectorSubcoreMesh(
        core_axis_name="c", subcore_axis_name="s", num_cores=NC, num_subcores=NS
    )


def _hist_body(src_hbm, dst_hbm, hist_hbm, srcv, dstv, hist_o, hist_i):
    cid = lax.axis_index("c")
    sid = lax.axis_index("s")
    wid = cid * NS + sid
    base = wid * EPT
    pltpu.sync_copy(src_hbm.at[pl.ds(base, EPT)], srcv)
    pltpu.sync_copy(dst_hbm.at[pl.ds(base, EPT)], dstv)

    zeros16 = jnp.zeros((16,), jnp.float32)

    def zbody(i, c):
        srcv_z = i * 16
        hist_o[pl.ds(srcv_z, 16)] = zeros16
        hist_i[pl.ds(srcv_z, 16)] = zeros16
        return c

    lax.fori_loop(0, N // 16, zbody, 0)

    ones16 = jnp.ones((16,), jnp.float32)

    def ebody(i, c):
        s = srcv[pl.ds(i * 16, 16)]
        plsc.addupdate_scatter(hist_o, [s], ones16)
        d = dstv[pl.ds(i * 16, 16)]
        plsc.addupdate_scatter(hist_i, [d], ones16)
        return c

    lax.fori_loop(0, EPT // 16, ebody, 0)

    pltpu.sync_copy(hist_o, hist_hbm.at[0, wid])
    pltpu.sync_copy(hist_i, hist_hbm.at[1, wid])


def _hist_call(src, dst):
    f = pl.kernel(
        _hist_body,
        out_type=jax.ShapeDtypeStruct((2, NW, N), jnp.float32),
        mesh=_mesh(),
        scratch_types=[
            pltpu.VMEM((EPT,), jnp.int32),
            pltpu.VMEM((EPT,), jnp.int32),
            pltpu.VMEM((N,), jnp.float32),
            pltpu.VMEM((N,), jnp.float32),
        ],
        compiler_params=pltpu.CompilerParams(needs_layout_passes=False),
    )
    return f(src, dst)


NBUF = 5


def _scatter_body(y_hbm, srcR, dstR, z_hbm, part_hbm, srcv, dstv, rows, *rest):
    sems = rest[:NBUF]
    acc = rest[NBUF]
    cid = lax.axis_index("c")
    sid = lax.axis_index("s")
    wid = cid * NS + sid
    r0 = sid * ROWS_PT
    pltpu.sync_copy(z_hbm.at[pl.ds(r0, ROWS_PT)], acc.at[pl.ds(r0, ROWS_PT)])
    pltpu.sync_copy(srcR.at[pl.ds(wid * NCH, NCH)], srcv)
    pltpu.sync_copy(dstR.at[pl.ds(wid * NCH, NCH)], dstv)
    plsc.subcore_barrier()

    for b in range(NBUF):
        pltpu.async_copy(y_hbm.at[srcv.at[b]], rows.at[b], sems[b])

    def body(g, c):
        for b in range(NBUF):
            j = g * NBUF + b
            pltpu.make_async_copy(y_hbm.at[srcv.at[j]], rows.at[b], sems[b]).wait()
            pltpu.sync_copy(rows.at[b], acc.at[dstv.at[j]], add=True)
            pltpu.async_copy(y_hbm.at[srcv.at[j + NBUF]], rows.at[b], sems[b])
        return c

    lax.fori_loop(0, NCH // NBUF - 1, body, 0)
    for b in range(NBUF):
        j = NCH - NBUF + b
        pltpu.make_async_copy(y_hbm.at[srcv.at[j]], rows.at[b], sems[b]).wait()
        pltpu.sync_copy(rows.at[b], acc.at[dstv.at[j]], add=True)

    plsc.subcore_barrier()
    pltpu.sync_copy(acc.at[pl.ds(r0, ROWS_PT)], part_hbm.at[cid, pl.ds(r0, ROWS_PT)])


def _scatter_call(y, srcR, dstR, zrows):
    f = pl.kernel(
        _scatter_body,
        out_type=jax.ShapeDtypeStruct((NC, N, H), jnp.float32),
        mesh=_mesh(),
        scratch_types=[
            pltpu.VMEM((NCH, CH), jnp.int32),
            pltpu.VMEM((NCH, CH), jnp.int32),
            pltpu.VMEM((NBUF, CH, H), jnp.float32),
        ]
        + [pltpu.SemaphoreType.DMA] * NBUF
        + [pltpu.VMEM_SHARED((N, H), jnp.float32)],
        compiler_params=pltpu.CompilerParams(
            needs_layout_passes=False, use_tc_tiling_on_sc=False
        ),
    )
    return f(y, srcR, dstR, zrows)


def _tc1_body(hist_ref, x_ref, w_ref, y_ref, nrm_ref):
    od = jnp.sum(hist_ref[0], axis=-1, keepdims=True)
    idg = jnp.sum(hist_ref[1], axis=-1, keepdims=True)
    onrm = lax.rsqrt(jnp.maximum(od, 1.0))
    inrm = lax.rsqrt(jnp.maximum(idg, 1.0))
    y = jnp.dot(x_ref[...], w_ref[...], preferred_element_type=jnp.float32)
    y_ref[...] = onrm * y
    nrm_ref[...] = jnp.concatenate([onrm, inrm], axis=1)


def _tc1_call(hist_t, x, w1):
    return pl.pallas_call(
        _tc1_body,
        grid=(N // BLK,),
        in_specs=[
            pl.BlockSpec((2, BLK, NW), lambda i: (0, i, 0)),
            pl.BlockSpec((BLK, D), lambda i: (i, 0)),
            pl.BlockSpec((D, H), lambda i: (0, 0)),
        ],
        out_specs=[
            pl.BlockSpec((BLK, H), lambda i: (i, 0)),
            pl.BlockSpec((BLK, 2), lambda i: (i, 0)),
        ],
        out_shape=[
            jax.ShapeDtypeStruct((N, H), jnp.float32),
            jax.ShapeDtypeStruct((N, 2), jnp.float32),
        ],
    )(hist_t, x, w1)


def _tc2_body(part_ref, nrm_ref, b_ref, w_ref, y_ref):
    agg = part_ref[0] + part_ref[1]
    inrm = nrm_ref[:, 1:2]
    onrm = nrm_ref[:, 0:1]
    h = jnp.maximum(agg * inrm + b_ref[...], 0.0)
    y_ref[...] = onrm * jnp.dot(h, w_ref[...], preferred_element_type=jnp.float32)


def _tc2_call(part, nrm, b1, w2):
    return pl.pallas_call(
        _tc2_body,
        grid=(N // BLK,),
        in_specs=[
            pl.BlockSpec((NC, BLK, H), lambda i: (0, i, 0)),
            pl.BlockSpec((BLK, 2), lambda i: (i, 0)),
            pl.BlockSpec((1, H), lambda i: (0, 0)),
            pl.BlockSpec((H, H), lambda i: (0, 0)),
        ],
        out_specs=pl.BlockSpec((BLK, H), lambda i: (i, 0)),
        out_shape=jax.ShapeDtypeStruct((N, H), jnp.float32),
    )(part, nrm, b1, w2)


def _tc3_body(part_ref, nrm_ref, b_ref, o_ref):
    agg = part_ref[0] + part_ref[1]
    inrm = nrm_ref[:, 1:2]
    o_ref[...] = jnp.maximum(agg * inrm + b_ref[...], 0.0)


def _tc3_call(part, nrm, b2):
    return pl.pallas_call(
        _tc3_body,
        grid=(N // BLK,),
        in_specs=[
            pl.BlockSpec((NC, BLK, H), lambda i: (0, i, 0)),
            pl.BlockSpec((BLK, 2), lambda i: (i, 0)),
            pl.BlockSpec((1, H), lambda i: (0, 0)),
        ],
        out_specs=pl.BlockSpec((BLK, H), lambda i: (i, 0)),
        out_shape=jax.ShapeDtypeStruct((N, H), jnp.float32),
    )(part, nrm, b2)


def kernel(features, edge_index, W1, b1, W2, b2):
    src = edge_index[0]
    dst = edge_index[1]
    srcR = src.reshape(E // CH, CH)
    dstR = dst.reshape(E // CH, CH)
    zrows = jnp.zeros((N, H), jnp.float32)

    hist = _hist_call(src, dst)
    hist_t = jnp.transpose(hist, (0, 2, 1))
    y1, nrm = _tc1_call(hist_t, features, W1)
    part1 = _scatter_call(y1, srcR, dstR, zrows)
    y2 = _tc2_call(part1, nrm, b1.reshape(1, H), W2)
    part2 = _scatter_call(y2, srcR, dstR, zrows)
    out = _tc3_call(part2, nrm, b2.reshape(1, H))
    return out


# speedup vs baseline: 14.8255x; 1.0257x over previous
import functools

import jax
import jax.numpy as jnp
from jax import lax
from jax.experimental import pallas as pl
from jax.experimental.pallas import tpu as pltpu
from jax.experimental.pallas import tpu_sc as plsc

N = 10000
E = 320000
D = 128
H = 128

NC = 2
NS = 16
NW = NC * NS
EPT = E // NW
CH = 40
NCH = EPT // CH
ROWS_PT = N // NS
BLK = 1000


def _mesh():
    return plsc.VectorSubcoreMesh(
        core_axis_name="c", subcore_axis_name="s", num_cores=NC, num_subcores=NS
    )


def _hist_body(src_hbm, dst_hbm, hist_hbm, srcv, dstv, hist_o, hist_i):
    cid = lax.axis_index("c")
    sid = lax.axis_index("s")
    wid = cid * NS + sid
    base = wid * EPT
    pltpu.sync_copy(src_hbm.at[pl.ds(base, EPT)], srcv)
    pltpu.sync_copy(dst_hbm.at[pl.ds(base, EPT)], dstv)

    zeros16 = jnp.zeros((16,), jnp.float32)

    def zbody(i, c):
        for u in range(5):
            o = (i * 5 + u) * 16
            hist_o[pl.ds(o, 16)] = zeros16
            hist_i[pl.ds(o, 16)] = zeros16
        return c

    lax.fori_loop(0, N // 80, zbody, 0)

    ones16 = jnp.ones((16,), jnp.float32)

    def ebody(i, c):
        for u in range(5):
            o = (i * 5 + u) * 16
            s = srcv[pl.ds(o, 16)]
            plsc.addupdate_scatter(hist_o, [s], ones16)
            d = dstv[pl.ds(o, 16)]
            plsc.addupdate_scatter(hist_i, [d], ones16)
        return c

    lax.fori_loop(0, EPT // 80, ebody, 0)

    pltpu.sync_copy(hist_o, hist_hbm.at[0, wid])
    pltpu.sync_copy(hist_i, hist_hbm.at[1, wid])


def _hist_call(src, dst):
    f = pl.kernel(
        _hist_body,
        out_type=jax.ShapeDtypeStruct((2, NW, N), jnp.float32),
        mesh=_mesh(),
        scratch_types=[
            pltpu.VMEM((EPT,), jnp.int32),
            pltpu.VMEM((EPT,), jnp.int32),
            pltpu.VMEM((N,), jnp.float32),
            pltpu.VMEM((N,), jnp.float32),
        ],
        compiler_params=pltpu.CompilerParams(
            needs_layout_passes=False, use_tc_tiling_on_sc=False
        ),
    )
    return f(src, dst)


NBUF = 5


def _scatter_body(y_hbm, srcR, dstR, part_hbm, srcv, dstv, rows, *rest):
    sems = rest[:NBUF]
    acc = rest[NBUF]
    cid = lax.axis_index("c")
    sid = lax.axis_index("s")
    wid = cid * NS + sid
    r0 = sid * ROWS_PT
    pltpu.sync_copy(srcR.at[pl.ds(wid * NCH, NCH)], srcv)
    pltpu.sync_copy(dstR.at[pl.ds(wid * NCH, NCH)], dstv)
    zeros16 = jnp.zeros((16,), jnp.float32)

    def zb(i, c):
        for u in range(H // 16):
            rows[0, i, pl.ds(u * 16, 16)] = zeros16
        return c

    lax.fori_loop(0, CH, zb, 0)
    for t in range(ROWS_PT // CH):
        pltpu.sync_copy(rows.at[0], acc.at[pl.ds(r0 + t * CH, CH)])
    pltpu.sync_copy(
        rows.at[0, pl.ds(0, ROWS_PT % CH)],
        acc.at[pl.ds(r0 + (ROWS_PT // CH) * CH, ROWS_PT % CH)],
    )
    plsc.subcore_barrier()

    for b in range(NBUF):
        pltpu.async_copy(y_hbm.at[srcv.at[b]], rows.at[b], sems[b])

    def body(g, c):
        for b in range(NBUF):
            j = g * NBUF + b
            pltpu.make_async_copy(y_hbm.at[srcv.at[j]], rows.at[b], sems[b]).wait()
            pltpu.sync_copy(rows.at[b], acc.at[dstv.at[j]], add=True)
            pltpu.async_copy(y_hbm.at[srcv.at[j + NBUF]], rows.at[b], sems[b])
        return c

    lax.fori_loop(0, NCH // NBUF - 1, body, 0)
    for b in range(NBUF):
        j = NCH - NBUF + b
        pltpu.make_async_copy(y_hbm.at[srcv.at[j]], rows.at[b], sems[b]).wait()
        pltpu.sync_copy(rows.at[b], acc.at[dstv.at[j]], add=True)

    plsc.subcore_barrier()
    pltpu.sync_copy(acc.at[pl.ds(r0, ROWS_PT)], part_hbm.at[cid, pl.ds(r0, ROWS_PT)])


def _scatter_call(y, srcR, dstR):
    f = pl.kernel(
        _scatter_body,
        out_type=jax.ShapeDtypeStruct((NC, N, H), jnp.float32),
        mesh=_mesh(),
        scratch_types=[
            pltpu.VMEM((NCH, CH), jnp.int32),
            pltpu.VMEM((NCH, CH), jnp.int32),
            pltpu.VMEM((NBUF, CH, H), jnp.float32),
        ]
        + [pltpu.SemaphoreType.DMA] * NBUF
        + [pltpu.VMEM_SHARED((N, H), jnp.float32)],
        compiler_params=pltpu.CompilerParams(
            needs_layout_passes=False, use_tc_tiling_on_sc=False
        ),
    )
    return f(y, srcR, dstR)


def _tc0_body(x_ref, w_ref, z_ref):
    z_ref[...] = jnp.dot(x_ref[...], w_ref[...], preferred_element_type=jnp.float32)


def _tc0_call(x, w1):
    return pl.pallas_call(
        _tc0_body,
        grid=(N // BLK,),
        in_specs=[
            pl.BlockSpec((BLK, D), lambda i: (i, 0)),
            pl.BlockSpec((D, H), lambda i: (0, 0)),
        ],
        out_specs=pl.BlockSpec((BLK, H), lambda i: (i, 0)),
        out_shape=jax.ShapeDtypeStruct((N, H), jnp.float32),
    )(x, w1)


def _tc1_body(hist_ref, z_ref, y_ref, nrm_ref):
    od = jnp.sum(hist_ref[0], axis=-1, keepdims=True)
    idg = jnp.sum(hist_ref[1], axis=-1, keepdims=True)
    onrm = lax.rsqrt(jnp.maximum(od, 1.0))
    inrm = lax.rsqrt(jnp.maximum(idg, 1.0))
    y_ref[...] = onrm * z_ref[...]
    nrm_ref[...] = jnp.concatenate([onrm, inrm], axis=1)


def _tc1_call(hist_t, z1):
    return pl.pallas_call(
        _tc1_body,
        grid=(N // BLK,),
        in_specs=[
            pl.BlockSpec((2, BLK, NW), lambda i: (0, i, 0)),
            pl.BlockSpec((BLK, H), lambda i: (i, 0)),
        ],
        out_specs=[
            pl.BlockSpec((BLK, H), lambda i: (i, 0)),
            pl.BlockSpec((BLK, 2), lambda i: (i, 0)),
        ],
        out_shape=[
            jax.ShapeDtypeStruct((N, H), jnp.float32),
            jax.ShapeDtypeStruct((N, 2), jnp.float32),
        ],
    )(hist_t, z1)


def _tc2_body(part_ref, nrm_ref, b_ref, w_ref, y_ref):
    agg = part_ref[0] + part_ref[1]
    inrm = nrm_ref[:, 1:2]
    onrm = nrm_ref[:, 0:1]
    h = jnp.maximum(agg * inrm + b_ref[...], 0.0)
    y_ref[...] = onrm * jnp.dot(h, w_ref[...], preferred_element_type=jnp.float32)


def _tc2_call(part, nrm, b1, w2):
    return pl.pallas_call(
        _tc2_body,
        grid=(N // BLK,),
        in_specs=[
            pl.BlockSpec((NC, BLK, H), lambda i: (0, i, 0)),
            pl.BlockSpec((BLK, 2), lambda i: (i, 0)),
            pl.BlockSpec((1, H), lambda i: (0, 0)),
            pl.BlockSpec((H, H), lambda i: (0, 0)),
        ],
        out_specs=pl.BlockSpec((BLK, H), lambda i: (i, 0)),
        out_shape=jax.ShapeDtypeStruct((N, H), jnp.float32),
    )(part, nrm, b1, w2)


def _tc3_body(part_ref, nrm_ref, b_ref, o_ref):
    agg = part_ref[0] + part_ref[1]
    inrm = nrm_ref[:, 1:2]
    o_ref[...] = jnp.maximum(agg * inrm + b_ref[...], 0.0)


def _tc3_call(part, nrm, b2):
    return pl.pallas_call(
        _tc3_body,
        grid=(N // BLK,),
        in_specs=[
            pl.BlockSpec((NC, BLK, H), lambda i: (0, i, 0)),
            pl.BlockSpec((BLK, 2), lambda i: (i, 0)),
            pl.BlockSpec((1, H), lambda i: (0, 0)),
        ],
        out_specs=pl.BlockSpec((BLK, H), lambda i: (i, 0)),
        out_shape=jax.ShapeDtypeStruct((N, H), jnp.float32),
    )(part, nrm, b2)


def kernel(features, edge_index, W1, b1, W2, b2):
    src = edge_index[0]
    dst = edge_index[1]
    srcR = src.reshape(E // CH, CH)
    dstR = dst.reshape(E // CH, CH)

    hist = _hist_call(src, dst)
    hist_t = jnp.transpose(hist, (0, 2, 1))
    z1 = _tc0_call(features, W1)
    y1, nrm = _tc1_call(hist_t, z1)
    part1 = _scatter_call(y1, srcR, dstR)
    y2 = _tc2_call(part1, nrm, b1.reshape(1, H), W2)
    part2 = _scatter_call(y2, srcR, dstR)
    out = _tc3_call(part2, nrm, b2.reshape(1, H))
    return out
